# baseline (speedup 1.0000x reference)
import functools

import jax
import jax.numpy as jnp
from jax import lax
from jax.experimental import pallas as pl
from jax.experimental.pallas import tpu as pltpu
from jax.experimental.pallas import tpu_sc as plsc

L = 16
ROWS = 32
T = 8
V = 100000
NBLK = V // L
U = 10
STEPS = NBLK // U
NCH = 5
NS = 16
SC_ROWS = 16
PER_SC = SC_ROWS // 2
IMAX = 2**31 - 1


def _sc_argmax_kernel(x_hbm, out_hbm, buf_v, res_v, sem):
    cid = lax.axis_index("c")
    sid = lax.axis_index("s")
    wid = cid * PER_SC + sid

    @pl.when(sid < PER_SC)
    def _():
        pltpu.async_copy(x_hbm.at[wid, T - 1], buf_v, sem).wait()

        mxs = [jnp.full((L,), -jnp.inf, jnp.float32) for _ in range(NCH)]
        ixs = [jnp.zeros((L,), jnp.int32) for _ in range(NCH)]

        def body(t, carry):
            mxs, ixs = list(carry[0]), list(carry[1])
            base = t * U
            for u in range(U):
                v = buf_v[pl.ds((base + u) * L, L)]
                blk = jnp.full((L,), base + u, jnp.int32)
                j = u % NCH
                pred = v > mxs[j]
                mxs[j] = jnp.maximum(mxs[j], v)
                ixs[j] = jnp.where(pred, blk, ixs[j])
            return tuple(mxs), tuple(ixs)

        mt, it_ = lax.fori_loop(0, STEPS, body, (tuple(mxs), tuple(ixs)))
        mxs, ixs = list(mt), list(it_)

        mx, ix = mxs[0], ixs[0]
        for j in range(1, NCH):
            pred = (mxs[j] > mx) | ((mxs[j] == mx) & (ixs[j] < ix))
            mx = jnp.where(pred, mxs[j], mx)
            ix = jnp.where(pred, ixs[j], ix)

        iota = lax.iota(jnp.int32, L)

        def shuffle(v, s):
            return v.at[iota ^ s].get(mode="promise_in_bounds")

        gi = ix * L + iota
        m = mx
        for s in (8, 4, 2, 1):
            m = jnp.maximum(m, shuffle(m, s))
        cand = jnp.where(mx == m, gi, IMAX)
        for s in (8, 4, 2, 1):
            cand = jnp.minimum(cand, shuffle(cand, s))
        res_v[...] = cand
        pltpu.sync_copy(res_v, out_hbm.at[wid])


def _tc_argmax_kernel(x_ref, out_ref):
    x = x_ref[...]
    m = jnp.max(x, axis=-1, keepdims=True)
    ii = lax.broadcasted_iota(jnp.int32, (8, V), 1)
    cand = jnp.where(x == m, ii, IMAX)
    idx = jnp.min(cand, axis=-1, keepdims=True)
    out_ref[...] = jnp.broadcast_to(idx, (8, 128))


def kernel(measure):
    mesh = plsc.VectorSubcoreMesh(core_axis_name="c", subcore_axis_name="s")
    sc_run = functools.partial(
        pl.kernel,
        mesh=mesh,
        out_type=jax.ShapeDtypeStruct((SC_ROWS, L), jnp.int32),
        scratch_types=[
            pltpu.VMEM((V,), jnp.float32),
            pltpu.VMEM((L,), jnp.int32),
            pltpu.SemaphoreType.DMA,
        ],
    )(_sc_argmax_kernel)
    sc_out = sc_run(measure)

    ntc = (ROWS - SC_ROWS) // 8
    xt = jax.lax.slice(
        measure, (SC_ROWS, T - 1, 0), (ROWS, T, V)
    ).reshape(ROWS - SC_ROWS, V)
    tc_out = pl.pallas_call(
        _tc_argmax_kernel,
        grid=(ntc,),
        in_specs=[pl.BlockSpec((8, V), lambda i: (i, 0))],
        out_specs=pl.BlockSpec((8, 128), lambda i: (i, 0)),
        out_shape=jax.ShapeDtypeStruct((ROWS - SC_ROWS, 128), jnp.int32),
    )(xt)

    return jnp.concatenate([sc_out[:, 0], tc_out[:, 0]])


# speedup vs baseline: 2.0028x; 2.0028x over previous
import functools

import jax
import jax.numpy as jnp
from jax import lax
from jax.experimental import pallas as pl
from jax.experimental.pallas import tpu as pltpu
from jax.experimental.pallas import tpu_sc as plsc

L = 16
ROWS = 32
T = 8
V = 100000
NBLK = V // L
U = 10
STEPS = NBLK // U
NCH = 5
NS = 16
SC_ROWS = 16
PER_SC = SC_ROWS // 2
IMAX = 2**31 - 1


def _sc_argmax_kernel(x_hbm, out_hbm, buf_v, res_v, sem):
    cid = lax.axis_index("c")
    sid = lax.axis_index("s")
    wid = cid * PER_SC + sid

    @pl.when(sid < PER_SC)
    def _():
        pltpu.async_copy(x_hbm.at[wid, T - 1], buf_v, sem).wait()

        mxs = [jnp.full((L,), -jnp.inf, jnp.float32) for _ in range(NCH)]
        ixs = [jnp.zeros((L,), jnp.int32) for _ in range(NCH)]

        def body(t, carry):
            mxs, ixs = list(carry[0]), list(carry[1])
            base = t * U
            for u in range(U):
                v = buf_v[pl.ds((base + u) * L, L)]
                blk = jnp.full((L,), base + u, jnp.int32)
                j = u % NCH
                pred = v > mxs[j]
                mxs[j] = jnp.maximum(mxs[j], v)
                ixs[j] = jnp.where(pred, blk, ixs[j])
            return tuple(mxs), tuple(ixs)

        mt, it_ = lax.fori_loop(0, STEPS, body, (tuple(mxs), tuple(ixs)))
        mxs, ixs = list(mt), list(it_)

        mx, ix = mxs[0], ixs[0]
        for j in range(1, NCH):
            pred = (mxs[j] > mx) | ((mxs[j] == mx) & (ixs[j] < ix))
            mx = jnp.where(pred, mxs[j], mx)
            ix = jnp.where(pred, ixs[j], ix)

        iota = lax.iota(jnp.int32, L)

        def shuffle(v, s):
            return v.at[iota ^ s].get(mode="promise_in_bounds")

        gi = ix * L + iota
        m = mx
        for s in (8, 4, 2, 1):
            m = jnp.maximum(m, shuffle(m, s))
        cand = jnp.where(mx == m, gi, IMAX)
        for s in (8, 4, 2, 1):
            cand = jnp.minimum(cand, shuffle(cand, s))
        res_v[...] = cand
        pltpu.sync_copy(res_v, out_hbm.at[wid])


def _tc_argmax_kernel(x_hbm, out_ref, buf, sem):
    g = pl.program_id(0)
    for s in range(8):
        pltpu.make_async_copy(
            x_hbm.at[SC_ROWS + g * 8 + s, T - 1], buf.at[s], sem
        ).start()
    for s in range(8):
        pltpu.make_async_copy(
            x_hbm.at[SC_ROWS + g * 8 + s, T - 1], buf.at[s], sem
        ).wait()
    x = buf[...]
    m = jnp.max(x, axis=-1, keepdims=True)
    ii = lax.broadcasted_iota(jnp.int32, (8, V), 1)
    cand = jnp.where(x == m, ii, IMAX)
    idx = jnp.min(cand, axis=-1, keepdims=True)
    out_ref[...] = jnp.broadcast_to(idx, (8, 128))


def kernel(measure):
    mesh = plsc.VectorSubcoreMesh(core_axis_name="c", subcore_axis_name="s")
    sc_run = functools.partial(
        pl.kernel,
        mesh=mesh,
        out_type=jax.ShapeDtypeStruct((SC_ROWS, L), jnp.int32),
        scratch_types=[
            pltpu.VMEM((V,), jnp.float32),
            pltpu.VMEM((L,), jnp.int32),
            pltpu.SemaphoreType.DMA,
        ],
    )(_sc_argmax_kernel)
    sc_out = sc_run(measure)

    ntc = (ROWS - SC_ROWS) // 8
    tc_out = pl.pallas_call(
        _tc_argmax_kernel,
        grid=(ntc,),
        in_specs=[pl.BlockSpec(memory_space=pl.ANY)],
        out_specs=pl.BlockSpec((8, 128), lambda i: (i, 0)),
        out_shape=jax.ShapeDtypeStruct((ROWS - SC_ROWS, 128), jnp.int32),
        scratch_shapes=[
            pltpu.VMEM((8, V), jnp.float32),
            pltpu.SemaphoreType.DMA,
        ],
    )(measure)

    return jnp.concatenate([sc_out[:, 0], tc_out[:, 0]])


# speedup vs baseline: 2.0446x; 1.0209x over previous
import functools

import jax
import jax.numpy as jnp
from jax import lax
from jax.experimental import pallas as pl
from jax.experimental.pallas import tpu as pltpu
from jax.experimental.pallas import tpu_sc as plsc

L = 16
ROWS = 32
T = 8
V = 100000
NBLK = V // L
U = 10
STEPS = NBLK // U
NCH = 5
NS = 16
IMAX = 2**31 - 1


def _argmax_kernel(x_hbm, out_hbm, buf_v, res_v, sem):
    cid = lax.axis_index("c")
    sid = lax.axis_index("s")
    wid = cid * NS + sid
    pltpu.async_copy(x_hbm.at[wid, T - 1], buf_v, sem).wait()

    mxs = [jnp.full((L,), -jnp.inf, jnp.float32) for _ in range(NCH)]
    ixs = [jnp.zeros((L,), jnp.int32) for _ in range(NCH)]

    def body(t, carry):
        mxs, ixs = list(carry[0]), list(carry[1])
        base = t * U
        for u in range(U):
            v = buf_v[pl.ds((base + u) * L, L)]
            blk = jnp.full((L,), base + u, jnp.int32)
            j = u % NCH
            pred = v > mxs[j]
            mxs[j] = jnp.maximum(mxs[j], v)
            ixs[j] = jnp.where(pred, blk, ixs[j])
        return tuple(mxs), tuple(ixs)

    mt, it_ = lax.fori_loop(0, STEPS, body, (tuple(mxs), tuple(ixs)))
    mxs, ixs = list(mt), list(it_)

    mx, ix = mxs[0], ixs[0]
    for j in range(1, NCH):
        pred = (mxs[j] > mx) | ((mxs[j] == mx) & (ixs[j] < ix))
        mx = jnp.where(pred, mxs[j], mx)
        ix = jnp.where(pred, ixs[j], ix)

    iota = lax.iota(jnp.int32, L)

    def shuffle(v, s):
        return v.at[iota ^ s].get(mode="promise_in_bounds")

    gi = ix * L + iota
    m = mx
    for s in (8, 4, 2, 1):
        m = jnp.maximum(m, shuffle(m, s))
    cand = jnp.where(mx == m, gi, IMAX)
    for s in (8, 4, 2, 1):
        cand = jnp.minimum(cand, shuffle(cand, s))
    res_v[...] = cand
    pltpu.sync_copy(res_v, out_hbm.at[wid])


def kernel(measure):
    mesh = plsc.VectorSubcoreMesh(core_axis_name="c", subcore_axis_name="s")
    run = functools.partial(
        pl.kernel,
        mesh=mesh,
        out_type=jax.ShapeDtypeStruct((ROWS, L), jnp.int32),
        scratch_types=[
            pltpu.VMEM((V,), jnp.float32),
            pltpu.VMEM((L,), jnp.int32),
            pltpu.SemaphoreType.DMA,
        ],
    )(_argmax_kernel)
    out = run(measure)
    return out[:, 0]
